# baseline (speedup 1.0000x reference)
import functools

import numpy as np
import jax
import jax.numpy as jnp
from jax.experimental import pallas as pl
from jax.experimental.pallas import tpu as pltpu

_S = 0.025
_ALPHA = 0.98
_DELTA = 2.0
_EPS = 1e-6

_BB = 8
_L = 128

_IDX = np.arange(_L)
_DIFF = _IDX[None, :] - _IDX[:, None]
_W_NP = np.where(_DIFF >= 0,
                 _S * (1.0 - _S) ** np.maximum(_DIFF, 0),
                 0.0).astype(np.float32)
_D_NP = np.zeros((_L, _L), np.float32)
_D_NP[_L - 1, :] = (1.0 - _S) ** (_IDX + 1.0)
_WD_NP = np.concatenate([_W_NP, _D_NP], axis=0)
_SQRT_DELTA = float(np.sqrt(_DELTA))


def _pcen_body(x_ref, wd_ref, o_ref, mprev_ref, *, t_total, n_t, rows):
    t = pl.program_id(1)

    @pl.when(t == 0)
    def _():
        mprev_ref[...] = jnp.zeros_like(mprev_ref)

    @pl.when(t == n_t - 1)
    def _():
        col = jax.lax.broadcasted_iota(jnp.int32, (1, 1, _L), 2)
        x_ref[...] = jnp.where(col < (t_total - t * _L), x_ref[...], 0.0)

    x = x_ref[...].reshape(rows, _L)
    z = jnp.concatenate([x, mprev_ref[...]], axis=1)
    m = jnp.dot(z, wd_ref[...], preferred_element_type=jnp.float32,
                precision=jax.lax.Precision.HIGHEST)
    mprev_ref[...] = m

    p = jnp.exp(-_ALPHA * jnp.log(_EPS + m))
    y = jnp.sqrt(x * p + _DELTA) - _SQRT_DELTA
    o_ref[...] = y.reshape(_BB, -1, _L)


def kernel(mel_power):
    B, C, T = mel_power.shape
    rows = _BB * C
    n_r = B // _BB
    n_t = pl.cdiv(T, _L)
    out = pl.pallas_call(
        functools.partial(_pcen_body, t_total=T, n_t=n_t, rows=rows),
        grid=(n_r, n_t),
        in_specs=[
            pl.BlockSpec((_BB, C, _L), lambda r, t: (r, 0, t)),
            pl.BlockSpec((2 * _L, _L), lambda r, t: (0, 0)),
        ],
        out_specs=pl.BlockSpec((_BB, C, _L), lambda r, t: (r, 0, t)),
        out_shape=jax.ShapeDtypeStruct((B, C, T), jnp.float32),
        scratch_shapes=[pltpu.VMEM((rows, _L), jnp.float32)],
        compiler_params=pltpu.CompilerParams(
            dimension_semantics=("parallel", "arbitrary"),
        ),
    )(mel_power, jnp.asarray(_WD_NP))
    return out


# speedup vs baseline: 1.1601x; 1.1601x over previous
import functools

import numpy as np
import jax
import jax.numpy as jnp
from jax.experimental import pallas as pl
from jax.experimental.pallas import tpu as pltpu

_S = 0.025
_ALPHA = 0.98
_DELTA = 2.0
_EPS = 1e-6

_BB = 8
_L = 128

_IDX = np.arange(_L)
_DIFF = _IDX[None, :] - _IDX[:, None]
_W_NP = np.where(_DIFF >= 0,
                 _S * (1.0 - _S) ** np.maximum(_DIFF, 0),
                 0.0).astype(np.float32)
_D_NP = np.zeros((_L, _L), np.float32)
_D_NP[_L - 1, :] = (1.0 - _S) ** (_IDX + 1.0)
_WD_NP = np.concatenate([_W_NP, _D_NP], axis=0)
_SQRT_DELTA = float(np.sqrt(_DELTA))


def _pcen_body(x_ref, wd_ref, o_ref, mprev_ref, *, t_total, n_t, rows):
    t = pl.program_id(1)

    @pl.when(t == 0)
    def _():
        mprev_ref[...] = jnp.zeros_like(mprev_ref)

    @pl.when(t == n_t - 1)
    def _():
        col = jax.lax.broadcasted_iota(jnp.int32, (1, 1, _L), 2)
        x_ref[...] = jnp.where(col < (t_total - t * _L), x_ref[...], 0.0)

    x = x_ref[...].reshape(rows, _L)
    z = jnp.concatenate([x, mprev_ref[...]], axis=1)
    m = jnp.dot(z, wd_ref[...], preferred_element_type=jnp.float32)
    mprev_ref[...] = m

    p = jnp.exp(-_ALPHA * jnp.log(_EPS + m))
    y = jnp.sqrt(x * p + _DELTA) - _SQRT_DELTA
    o_ref[...] = y.reshape(_BB, -1, _L)


def kernel(mel_power):
    B, C, T = mel_power.shape
    rows = _BB * C
    n_r = B // _BB
    n_t = pl.cdiv(T, _L)
    out = pl.pallas_call(
        functools.partial(_pcen_body, t_total=T, n_t=n_t, rows=rows),
        grid=(n_r, n_t),
        in_specs=[
            pl.BlockSpec((_BB, C, _L), lambda r, t: (r, 0, t)),
            pl.BlockSpec((2 * _L, _L), lambda r, t: (0, 0)),
        ],
        out_specs=pl.BlockSpec((_BB, C, _L), lambda r, t: (r, 0, t)),
        out_shape=jax.ShapeDtypeStruct((B, C, T), jnp.float32),
        scratch_shapes=[pltpu.VMEM((rows, _L), jnp.float32)],
        compiler_params=pltpu.CompilerParams(
            dimension_semantics=("parallel", "arbitrary"),
        ),
    )(mel_power, jnp.asarray(_WD_NP))
    return out
